# baseline (speedup 1.0000x reference)
import jax
import jax.numpy as jnp
from jax import lax
from jax.experimental import pallas as pl
from jax.experimental.pallas import tpu as pltpu
from jax.experimental.pallas import tpu_sc as plsc

B = 16384
EMBED = 32
NEG = 8
NC = 2
NS = 16
NW = NC * NS
N = 1000000
NALIGNED = 999936
CH = 512
NCHUNKS1 = NALIGNED // CH
CPW = NCHUNKS1 // NW
T0 = NCHUNKS1 * CH * EMBED
TOTW = T0 + (N - NALIGNED) * EMBED
BPW = B // NW
C = 64
NCHUNK2 = BPW // C
H = 16


def _stream_body(u_t, i_t, u_tail, i_tail, u_pack, i_pack,
                 chunk, tailv, sem):
    wid = lax.axis_index("s") * NC + lax.axis_index("c")

    for tbl, tail, pack in ((u_t, u_tail, u_pack), (i_t, i_tail, i_pack)):
        def cb(c, _):
            g = wid * CPW + c
            pltpu.sync_copy(tbl.at[:, pl.ds(g * CH, CH)], chunk)
            base = pl.multiple_of(g * CH * EMBED, 512)
            for d in range(EMBED):
                pltpu.sync_copy(chunk.at[d], pack.at[pl.ds(base + d * CH, CH)])
            return 0

        lax.fori_loop(0, CPW, cb, 0)

        @pl.when(wid == 0)
        def _():
            pltpu.sync_copy(tail, tailv)
            for dd in range(4):
                pltpu.sync_copy(tailv.at[dd],
                                pack.at[pl.ds(T0 + dd * 512, 512)])


_stream = pl.kernel(
    _stream_body,
    out_type=(
        jax.ShapeDtypeStruct((TOTW,), jnp.float32),
        jax.ShapeDtypeStruct((TOTW,), jnp.float32),
    ),
    mesh=plsc.VectorSubcoreMesh(
        core_axis_name="c", subcore_axis_name="s",
        num_cores=NC, num_subcores=NS),
    scratch_types=[
        pltpu.VMEM((EMBED, CH), jnp.float32),
        pltpu.VMEM((4, 512), jnp.float32),
        pltpu.SemaphoreType.DMA,
    ],
    compiler_params=pltpu.CompilerParams(needs_layout_passes=False),
)


def _mf_body(u_pack, i_pack, ua_hbm, ia_hbm, na_hbm,
             pos_hbm, neg_hbm,
             ua, ia, na, uv, iv, nv, pos_buf, neg_buf, sem):
    wid = lax.axis_index("s") * NC + lax.axis_index("c")
    base = wid * BPW
    lanes = lax.broadcasted_iota(jnp.int32, (H,), 0)
    last = lanes == (H - 1)

    def chunk_body(c, _):
        cbase = base + c * C
        pltpu.sync_copy(ua_hbm.at[pl.ds(cbase * EMBED, C * EMBED)], ua)
        pltpu.sync_copy(ia_hbm.at[pl.ds(cbase * EMBED, C * EMBED)], ia)
        pltpu.sync_copy(
            na_hbm.at[pl.ds(cbase * NEG * EMBED, C * NEG * EMBED)], na)
        copies = []
        for k in range(C * EMBED // 128):
            copies.append(pltpu.async_copy(
                u_pack.at[ua.at[pl.ds(k * 128, 128)]],
                uv.at[pl.ds(k * 128, 128)], sem))
            copies.append(pltpu.async_copy(
                i_pack.at[ia.at[pl.ds(k * 128, 128)]],
                iv.at[pl.ds(k * 128, 128)], sem))
        for k in range(C * NEG * EMBED // 128):
            copies.append(pltpu.async_copy(
                i_pack.at[na.at[pl.ds(k * 128, 128)]],
                nv.at[pl.ds(k * 128, 128)], sem))
        for cp in copies:
            cp.wait()

        def elem_body(e, _):
            u0 = uv[pl.ds(e * EMBED, H)]
            u1 = uv[pl.ds(e * EMBED + H, H)]
            i0 = iv[pl.ds(e * EMBED, H)]
            i1 = iv[pl.ds(e * EMBED + H, H)]
            ps = plsc.cumsum(u0 * i0 + u1 * i1)
            plsc.store_scatter(
                pos_buf, [jnp.full((H,), c * C + e, jnp.int32)], ps,
                mask=last)
            for j in range(NEG):
                r = (e * NEG + j) * EMBED
                n0 = nv[pl.ds(r, H)]
                n1 = nv[pl.ds(r + H, H)]
                ns = plsc.cumsum(u0 * n0 + u1 * n1)
                plsc.store_scatter(
                    neg_buf,
                    [jnp.full((H,), (c * C + e) * NEG + j, jnp.int32)], ns,
                    mask=last)
            return 0

        lax.fori_loop(0, C, elem_body, 0)
        return 0

    lax.fori_loop(0, NCHUNK2, chunk_body, 0)
    pltpu.sync_copy(pos_buf, pos_hbm.at[pl.ds(base, BPW)])
    pltpu.sync_copy(neg_buf, neg_hbm.at[pl.ds(base * NEG, BPW * NEG)])


_mf = pl.kernel(
    _mf_body,
    out_type=(
        jax.ShapeDtypeStruct((B,), jnp.float32),
        jax.ShapeDtypeStruct((B * NEG,), jnp.float32),
    ),
    mesh=plsc.VectorSubcoreMesh(
        core_axis_name="c", subcore_axis_name="s",
        num_cores=NC, num_subcores=NS),
    scratch_types=[
        pltpu.VMEM((C * EMBED,), jnp.int32),
        pltpu.VMEM((C * EMBED,), jnp.int32),
        pltpu.VMEM((C * NEG * EMBED,), jnp.int32),
        pltpu.VMEM((C * EMBED,), jnp.float32),
        pltpu.VMEM((C * EMBED,), jnp.float32),
        pltpu.VMEM((C * NEG * EMBED,), jnp.float32),
        pltpu.VMEM((BPW,), jnp.float32),
        pltpu.VMEM((BPW * NEG,), jnp.float32),
        pltpu.SemaphoreType.DMA,
    ],
    compiler_params=pltpu.CompilerParams(
        needs_layout_passes=False, use_tc_tiling_on_sc=False),
)


def _addrs(idx):
    d = jnp.arange(EMBED, dtype=jnp.int32)[None, :]
    return (d * N + idx[:, None]).reshape(-1)


def kernel(user_embeds, item_embeds, users, items, items_neg):
    users = users.astype(jnp.int32)
    items = items.astype(jnp.int32)
    neg_flat = items_neg.astype(jnp.int32).reshape(B * NEG)
    ua = _addrs(users)
    ia = _addrs(items)
    na = _addrs(neg_flat)
    u_pack = user_embeds.T.reshape(N * EMBED)
    i_pack = item_embeds.T.reshape(N * EMBED)
    pos, neg = _mf(u_pack, i_pack, ua, ia, na)
    return pos, neg.reshape(B, NEG)


# speedup vs baseline: 1.0001x; 1.0001x over previous
import jax
import jax.numpy as jnp
from jax import lax
from jax.experimental import pallas as pl
from jax.experimental.pallas import tpu as pltpu
from jax.experimental.pallas import tpu_sc as plsc

B = 16384
EMBED = 32
NEG = 8
NC = 2
NS = 16
NW = NC * NS
N = 1000000
BPW = B // NW
C = 64
NCHUNK2 = BPW // C
H = 16


def _mf_body(u_pack, i_pack, ua_hbm, ia_hbm, na_hbm,
             pos_hbm, neg_hbm,
             ua, ia, na, uv, iv, nv, pos_buf, neg_buf, sem):
    wid = lax.axis_index("s") * NC + lax.axis_index("c")
    base = wid * BPW
    lanes = lax.broadcasted_iota(jnp.int32, (H,), 0)
    last = lanes == (H - 1)

    def chunk_body(c, _):
        cbase = base + c * C
        pltpu.sync_copy(ua_hbm.at[pl.ds(cbase * EMBED, C * EMBED)], ua)
        pltpu.sync_copy(ia_hbm.at[pl.ds(cbase * EMBED, C * EMBED)], ia)
        pltpu.sync_copy(
            na_hbm.at[pl.ds(cbase * NEG * EMBED, C * NEG * EMBED)], na)
        copies = [
            pltpu.async_copy(u_pack.at[ua], uv, sem),
            pltpu.async_copy(i_pack.at[ia], iv, sem),
            pltpu.async_copy(i_pack.at[na], nv, sem),
        ]
        for cp in copies:
            cp.wait()

        def elem_body(e, _):
            u0 = uv[pl.ds(e * EMBED, H)]
            u1 = uv[pl.ds(e * EMBED + H, H)]
            i0 = iv[pl.ds(e * EMBED, H)]
            i1 = iv[pl.ds(e * EMBED + H, H)]
            ps = plsc.cumsum(u0 * i0 + u1 * i1)
            plsc.store_scatter(
                pos_buf, [jnp.full((H,), c * C + e, jnp.int32)], ps,
                mask=last)
            for j in range(NEG):
                r = (e * NEG + j) * EMBED
                n0 = nv[pl.ds(r, H)]
                n1 = nv[pl.ds(r + H, H)]
                ns = plsc.cumsum(u0 * n0 + u1 * n1)
                plsc.store_scatter(
                    neg_buf,
                    [jnp.full((H,), (c * C + e) * NEG + j, jnp.int32)], ns,
                    mask=last)
            return 0

        lax.fori_loop(0, C, elem_body, 0)
        return 0

    lax.fori_loop(0, NCHUNK2, chunk_body, 0)
    pltpu.sync_copy(pos_buf, pos_hbm.at[pl.ds(base, BPW)])
    pltpu.sync_copy(neg_buf, neg_hbm.at[pl.ds(base * NEG, BPW * NEG)])


_mf = pl.kernel(
    _mf_body,
    out_type=(
        jax.ShapeDtypeStruct((B,), jnp.float32),
        jax.ShapeDtypeStruct((B * NEG,), jnp.float32),
    ),
    mesh=plsc.VectorSubcoreMesh(
        core_axis_name="c", subcore_axis_name="s",
        num_cores=NC, num_subcores=NS),
    scratch_types=[
        pltpu.VMEM((C * EMBED,), jnp.int32),
        pltpu.VMEM((C * EMBED,), jnp.int32),
        pltpu.VMEM((C * NEG * EMBED,), jnp.int32),
        pltpu.VMEM((C * EMBED,), jnp.float32),
        pltpu.VMEM((C * EMBED,), jnp.float32),
        pltpu.VMEM((C * NEG * EMBED,), jnp.float32),
        pltpu.VMEM((BPW,), jnp.float32),
        pltpu.VMEM((BPW * NEG,), jnp.float32),
        pltpu.SemaphoreType.DMA,
    ],
    compiler_params=pltpu.CompilerParams(
        needs_layout_passes=False, use_tc_tiling_on_sc=False),
)


def _addrs(idx):
    d = jnp.arange(EMBED, dtype=jnp.int32)[None, :]
    return (d * N + idx[:, None]).reshape(-1)


def kernel(user_embeds, item_embeds, users, items, items_neg):
    users = users.astype(jnp.int32)
    items = items.astype(jnp.int32)
    neg_flat = items_neg.astype(jnp.int32).reshape(B * NEG)
    ua = _addrs(users)
    ia = _addrs(items)
    na = _addrs(neg_flat)
    u_pack = user_embeds.T.reshape(N * EMBED)
    i_pack = item_embeds.T.reshape(N * EMBED)
    pos, neg = _mf(u_pack, i_pack, ua, ia, na)
    return pos, neg.reshape(B, NEG)
